# Initial kernel scaffold; baseline (speedup 1.0000x reference)
import jax
import jax.numpy as jnp
from jax.experimental import pallas as pl


def kernel(cat0, cat1, cat2, cat3, cat4, cat5, cat6, cat7, cat8, cat9, cat10, cat11, cat12, cat13, cat14, cat15, cat16, cat17, cat18, cat19, cat20, cat21, cat22, cat23, cat24, cat25, table_cat0, table_cat1, table_cat2, table_cat3, table_cat4, table_cat5, table_cat6, table_cat7, table_cat8, table_cat9, table_cat10, table_cat11, table_cat12, table_cat13, table_cat14, table_cat15, table_cat16, table_cat17, table_cat18, table_cat19, table_cat20, table_cat21, table_cat22, table_cat23, table_cat24, table_cat25):
    raise NotImplementedError("write your pallas kernel here")


# speedup vs baseline: 3.8245x; 3.8245x over previous
import functools

import jax
import jax.numpy as jnp
from jax import lax
from jax.experimental import pallas as pl
from jax.experimental.pallas import tpu as pltpu
from jax.experimental.pallas import tpu_sc as plsc

B = 4096
VOCAB = 1000
DIM = 128
NCOLS = 26

_info = plsc.get_sparse_core_info()
_NC = _info.num_cores
_NS = _info.num_subcores
_NW = _NC * _NS
_BPW = B // _NW

_mesh = plsc.VectorSubcoreMesh(core_axis_name="c", subcore_axis_name="s")


@functools.partial(
    pl.kernel,
    mesh=_mesh,
    out_type=[jax.ShapeDtypeStruct((B, DIM), jnp.float32)] * NCOLS,
    scratch_types=[
        pltpu.VMEM((_BPW,), jnp.int32),
        pltpu.VMEM((_BPW, DIM), jnp.float32),
        pltpu.SemaphoreType.DMA,
    ],
)
def _gather_all(*refs):
    idx_refs = refs[:NCOLS]
    tab_refs = refs[NCOLS:2 * NCOLS]
    out_refs = refs[2 * NCOLS:3 * NCOLS]
    idx_v, rows_v, sem = refs[3 * NCOLS:]
    wid = lax.axis_index("s") * _NC + lax.axis_index("c")
    base = wid * _BPW
    for c in range(NCOLS):
        pltpu.sync_copy(idx_refs[c].at[pl.ds(base, _BPW)], idx_v)
        pltpu.async_copy(tab_refs[c].at[idx_v], rows_v, sem).wait()
        pltpu.sync_copy(rows_v, out_refs[c].at[pl.ds(base, _BPW)])


def kernel(cat0, cat1, cat2, cat3, cat4, cat5, cat6, cat7, cat8, cat9,
           cat10, cat11, cat12, cat13, cat14, cat15, cat16, cat17, cat18,
           cat19, cat20, cat21, cat22, cat23, cat24, cat25,
           table_cat0, table_cat1, table_cat2, table_cat3, table_cat4,
           table_cat5, table_cat6, table_cat7, table_cat8, table_cat9,
           table_cat10, table_cat11, table_cat12, table_cat13, table_cat14,
           table_cat15, table_cat16, table_cat17, table_cat18, table_cat19,
           table_cat20, table_cat21, table_cat22, table_cat23, table_cat24,
           table_cat25):
    cats = (cat0, cat1, cat2, cat3, cat4, cat5, cat6, cat7, cat8, cat9,
            cat10, cat11, cat12, cat13, cat14, cat15, cat16, cat17, cat18,
            cat19, cat20, cat21, cat22, cat23, cat24, cat25)
    tabs = (table_cat0, table_cat1, table_cat2, table_cat3, table_cat4,
            table_cat5, table_cat6, table_cat7, table_cat8, table_cat9,
            table_cat10, table_cat11, table_cat12, table_cat13, table_cat14,
            table_cat15, table_cat16, table_cat17, table_cat18, table_cat19,
            table_cat20, table_cat21, table_cat22, table_cat23, table_cat24,
            table_cat25)
    idxs = tuple(c.reshape(B) for c in cats)
    outs = _gather_all(*idxs, *tabs)
    return tuple(o.reshape(B, 1, DIM) for o in outs)


# speedup vs baseline: 5.4135x; 1.4155x over previous
import functools

import jax
import jax.numpy as jnp
from jax import lax
from jax.experimental import pallas as pl
from jax.experimental.pallas import tpu as pltpu
from jax.experimental.pallas import tpu_sc as plsc

B = 4096
VOCAB = 1000
DIM = 128
NCOLS = 26

_info = plsc.get_sparse_core_info()
_NC = _info.num_cores
_NS = _info.num_subcores
_NW = _NC * _NS
_BPW = B // _NW

_mesh = plsc.VectorSubcoreMesh(core_axis_name="c", subcore_axis_name="s")


@functools.partial(
    pl.kernel,
    mesh=_mesh,
    out_type=[jax.ShapeDtypeStruct((B, DIM), jnp.float32)] * NCOLS,
    scratch_types=[
        pltpu.VMEM((NCOLS, _BPW), jnp.int32),
        pltpu.VMEM((_BPW, DIM), jnp.float32),
        pltpu.VMEM((_BPW, DIM), jnp.float32),
        pltpu.SemaphoreType.DMA,
        pltpu.SemaphoreType.DMA,
        pltpu.SemaphoreType.DMA,
        pltpu.SemaphoreType.DMA,
        pltpu.SemaphoreType.DMA,
    ],
)
def _gather_all(*refs):
    idx_refs = refs[:NCOLS]
    tab_refs = refs[NCOLS:2 * NCOLS]
    out_refs = refs[2 * NCOLS:3 * NCOLS]
    idx_all, rows0, rows1, sem_i, sg0, sg1, so0, so1 = refs[3 * NCOLS:]
    rows = (rows0, rows1)
    sem_g = (sg0, sg1)
    sem_o = (so0, so1)
    wid = lax.axis_index("s") * _NC + lax.axis_index("c")
    base = wid * _BPW
    idx_cp = [pltpu.async_copy(idx_refs[c].at[pl.ds(base, _BPW)],
                               idx_all.at[c], sem_i)
              for c in range(NCOLS)]
    for cp in idx_cp:
        cp.wait()
    g = [None] * NCOLS
    o = [None] * NCOLS
    g[0] = pltpu.async_copy(tab_refs[0].at[idx_all.at[0]], rows[0], sem_g[0])
    for c in range(NCOLS):
        b = c % 2
        nb = (c + 1) % 2
        if c + 1 < NCOLS:
            if c >= 1:
                o[c - 1].wait()
            g[c + 1] = pltpu.async_copy(tab_refs[c + 1].at[idx_all.at[c + 1]],
                                        rows[nb], sem_g[nb])
        g[c].wait()
        o[c] = pltpu.async_copy(rows[b], out_refs[c].at[pl.ds(base, _BPW)],
                                sem_o[b])
    o[NCOLS - 2].wait()
    o[NCOLS - 1].wait()


def kernel(cat0, cat1, cat2, cat3, cat4, cat5, cat6, cat7, cat8, cat9,
           cat10, cat11, cat12, cat13, cat14, cat15, cat16, cat17, cat18,
           cat19, cat20, cat21, cat22, cat23, cat24, cat25,
           table_cat0, table_cat1, table_cat2, table_cat3, table_cat4,
           table_cat5, table_cat6, table_cat7, table_cat8, table_cat9,
           table_cat10, table_cat11, table_cat12, table_cat13, table_cat14,
           table_cat15, table_cat16, table_cat17, table_cat18, table_cat19,
           table_cat20, table_cat21, table_cat22, table_cat23, table_cat24,
           table_cat25):
    cats = (cat0, cat1, cat2, cat3, cat4, cat5, cat6, cat7, cat8, cat9,
            cat10, cat11, cat12, cat13, cat14, cat15, cat16, cat17, cat18,
            cat19, cat20, cat21, cat22, cat23, cat24, cat25)
    tabs = (table_cat0, table_cat1, table_cat2, table_cat3, table_cat4,
            table_cat5, table_cat6, table_cat7, table_cat8, table_cat9,
            table_cat10, table_cat11, table_cat12, table_cat13, table_cat14,
            table_cat15, table_cat16, table_cat17, table_cat18, table_cat19,
            table_cat20, table_cat21, table_cat22, table_cat23, table_cat24,
            table_cat25)
    idxs = tuple(c.reshape(B) for c in cats)
    outs = _gather_all(*idxs, *tabs)
    return tuple(o.reshape(B, 1, DIM) for o in outs)


# speedup vs baseline: 5.7168x; 1.0560x over previous
import functools

import jax
import jax.numpy as jnp
from jax import lax
from jax.experimental import pallas as pl
from jax.experimental.pallas import tpu as pltpu
from jax.experimental.pallas import tpu_sc as plsc

B = 4096
VOCAB = 1000
DIM = 128
NCOLS = 26

_info = plsc.get_sparse_core_info()
_NC = _info.num_cores
_NS = _info.num_subcores
_NW = _NC * _NS
_BPW = B // _NW

_mesh = plsc.VectorSubcoreMesh(core_axis_name="c", subcore_axis_name="s")


_NBUF = 4


@functools.partial(
    pl.kernel,
    mesh=_mesh,
    out_type=[jax.ShapeDtypeStruct((B, DIM), jnp.float32)] * NCOLS,
    scratch_types=(
        [pltpu.VMEM((NCOLS, _BPW), jnp.int32)]
        + [pltpu.VMEM((_BPW, DIM), jnp.float32)] * _NBUF
        + [pltpu.SemaphoreType.DMA] * (1 + 2 * _NBUF)
    ),
)
def _gather_all(*refs):
    idx_refs = refs[:NCOLS]
    tab_refs = refs[NCOLS:2 * NCOLS]
    out_refs = refs[2 * NCOLS:3 * NCOLS]
    scratch = refs[3 * NCOLS:]
    idx_all = scratch[0]
    rows = scratch[1:1 + _NBUF]
    sem_i = scratch[1 + _NBUF]
    sem_g = scratch[2 + _NBUF:2 + 2 * _NBUF]
    sem_o = scratch[2 + 2 * _NBUF:2 + 3 * _NBUF]
    wid = lax.axis_index("s") * _NC + lax.axis_index("c")
    base = wid * _BPW
    idx_cp = [pltpu.async_copy(idx_refs[c].at[pl.ds(base, _BPW)],
                               idx_all.at[c], sem_i)
              for c in range(NCOLS)]
    for cp in idx_cp:
        cp.wait()
    g = [None] * NCOLS
    o = [None] * NCOLS
    for j in range(min(_NBUF - 1, NCOLS)):
        g[j] = pltpu.async_copy(tab_refs[j].at[idx_all.at[j]],
                                rows[j], sem_g[j])
    for c in range(NCOLS):
        b = c % _NBUF
        if c + _NBUF - 1 < NCOLS:
            nb = (c + _NBUF - 1) % _NBUF
            if c >= 1:
                o[c - 1].wait()
            g[c + _NBUF - 1] = pltpu.async_copy(
                tab_refs[c + _NBUF - 1].at[idx_all.at[c + _NBUF - 1]],
                rows[nb], sem_g[nb])
        g[c].wait()
        o[c] = pltpu.async_copy(rows[b], out_refs[c].at[pl.ds(base, _BPW)],
                                sem_o[b])
    for c in range(max(0, NCOLS - _NBUF), NCOLS):
        o[c].wait()


def kernel(cat0, cat1, cat2, cat3, cat4, cat5, cat6, cat7, cat8, cat9,
           cat10, cat11, cat12, cat13, cat14, cat15, cat16, cat17, cat18,
           cat19, cat20, cat21, cat22, cat23, cat24, cat25,
           table_cat0, table_cat1, table_cat2, table_cat3, table_cat4,
           table_cat5, table_cat6, table_cat7, table_cat8, table_cat9,
           table_cat10, table_cat11, table_cat12, table_cat13, table_cat14,
           table_cat15, table_cat16, table_cat17, table_cat18, table_cat19,
           table_cat20, table_cat21, table_cat22, table_cat23, table_cat24,
           table_cat25):
    cats = (cat0, cat1, cat2, cat3, cat4, cat5, cat6, cat7, cat8, cat9,
            cat10, cat11, cat12, cat13, cat14, cat15, cat16, cat17, cat18,
            cat19, cat20, cat21, cat22, cat23, cat24, cat25)
    tabs = (table_cat0, table_cat1, table_cat2, table_cat3, table_cat4,
            table_cat5, table_cat6, table_cat7, table_cat8, table_cat9,
            table_cat10, table_cat11, table_cat12, table_cat13, table_cat14,
            table_cat15, table_cat16, table_cat17, table_cat18, table_cat19,
            table_cat20, table_cat21, table_cat22, table_cat23, table_cat24,
            table_cat25)
    idxs = tuple(c.reshape(B) for c in cats)
    outs = _gather_all(*idxs, *tabs)
    return tuple(o.reshape(B, 1, DIM) for o in outs)
